# baseline (speedup 1.0000x reference)
import functools

import jax
import jax.numpy as jnp
from jax import lax
from jax.experimental import pallas as pl
from jax.experimental.pallas import tpu as pltpu
from jax.experimental.pallas import tpu_sc as plsc

B = 16384
D = 64
NC = 2
NS = 16
NW = NC * NS
BPW = B // NW
CHUNK = 128
NCHUNK = BPW // CHUNK
BLK = 16
NBLK = BPW // BLK


def _nrsqrt(y):
    i = lax.bitcast_convert_type(y, jnp.int32)
    i = jnp.int32(0x5F3759DF) - lax.shift_right_arithmetic(i, 1)
    x = lax.bitcast_convert_type(i, jnp.float32)
    for _ in range(3):
        x = x * (1.5 - 0.5 * y * x * x)
    return x


def _gmf_body(job_hbm, geek_hbm, jemb_hbm, gemb_hbm, w_hbm, out_hbm,
              idx_j, idx_g, rows_j, rows_g, w_v, out_v, sem):
    wid = lax.axis_index("s") * NC + lax.axis_index("c")
    base = wid * BPW

    pltpu.sync_copy(w_hbm, w_v)
    for c in range(NCHUNK):
        pltpu.sync_copy(job_hbm.at[pl.ds(base + c * CHUNK, CHUNK)], idx_j.at[c])
        pltpu.sync_copy(geek_hbm.at[pl.ds(base + c * CHUNK, CHUNK)], idx_g.at[c])

    copies = []
    for c in range(NCHUNK):
        copies.append(pltpu.async_copy(
            jemb_hbm.at[idx_j.at[c]], rows_j.at[pl.ds(c * CHUNK, CHUNK)], sem))
        copies.append(pltpu.async_copy(
            gemb_hbm.at[idx_g.at[c]], rows_g.at[pl.ds(c * CHUNK, CHUNK)], sem))
    for cp in copies:
        cp.wait()

    wv = [w_v[pl.ds(c * 16, 16)] for c in range(D // 16)]
    lane = lax.iota(jnp.int32, 16)
    perms = [jnp.bitwise_xor(lane, s) for s in (8, 4, 2, 1)]

    gdn = lax.GatherDimensionNumbers(
        offset_dims=(), collapsed_slice_dims=(0,), start_index_map=(0,))

    def lanesum(v):
        for p in perms:
            v = v + lax.gather(v, p[:, None], gdn, (1,),
                               mode=lax.GatherScatterMode.PROMISE_IN_BOUNDS)
        return v

    def block(blk, carry):
        row0 = blk * BLK
        vx = jnp.zeros((16,), jnp.float32)
        vjj = jnp.zeros((16,), jnp.float32)
        vgg = jnp.zeros((16,), jnp.float32)
        for r in range(BLK):
            row = row0 + r
            px = jnp.zeros((16,), jnp.float32)
            pjj = jnp.zeros((16,), jnp.float32)
            pgg = jnp.zeros((16,), jnp.float32)
            for c in range(D // 16):
                jv = rows_j[row, pl.ds(c * 16, 16)]
                gv = rows_g[row, pl.ds(c * 16, 16)]
                px = px + jv * gv * wv[c]
                pjj = pjj + jv * jv
                pgg = pgg + gv * gv
            m = lane == r
            vx = jnp.where(m, lanesum(px), vx)
            vjj = jnp.where(m, lanesum(pjj), vjj)
            vgg = jnp.where(m, lanesum(pgg), vgg)
        out_v[pl.ds(row0, BLK)] = vx * _nrsqrt(vjj * vgg)
        return carry

    lax.fori_loop(0, NBLK, block, 0)
    pltpu.sync_copy(out_v, out_hbm.at[pl.ds(base, BPW)])


@functools.partial(jax.jit, static_argnums=())
def _gmf(job_f, geek_f, job_emb, geek_emb, w_f):
    mesh = plsc.VectorSubcoreMesh(core_axis_name="c", subcore_axis_name="s")
    run = functools.partial(
        pl.kernel,
        out_type=jax.ShapeDtypeStruct((B,), jnp.float32),
        mesh=mesh,
        scratch_types=[
            pltpu.VMEM((NCHUNK, CHUNK), jnp.int32),
            pltpu.VMEM((NCHUNK, CHUNK), jnp.int32),
            pltpu.VMEM((BPW, D), jnp.float32),
            pltpu.VMEM((BPW, D), jnp.float32),
            pltpu.VMEM((D,), jnp.float32),
            pltpu.VMEM((BPW,), jnp.float32),
            pltpu.SemaphoreType.DMA,
        ],
        compiler_params=pltpu.CompilerParams(use_tc_tiling_on_sc=False),
    )(_gmf_body)
    return run(job_f, geek_f, job_emb, geek_emb, w_f)


def kernel(job, geek, job_emb, geek_emb, W):
    job_f = job.reshape(-1).astype(jnp.int32)
    geek_f = geek.reshape(-1).astype(jnp.int32)
    w_f = W.reshape(-1)
    out = _gmf(job_f, geek_f, job_emb, geek_emb, w_f)
    return out.reshape(B, 1)


# speedup vs baseline: 1.5717x; 1.5717x over previous
import functools

import jax
import jax.numpy as jnp
from jax import lax
from jax.experimental import pallas as pl
from jax.experimental.pallas import tpu as pltpu
from jax.experimental.pallas import tpu_sc as plsc

B = 16384
D = 64
NC = 2
NS = 16
NW = NC * NS
BPW = B // NW
NPASS = 2
PB = BPW // NPASS
BLK = 16


def _nrsqrt(y):
    i = lax.bitcast_convert_type(y, jnp.int32)
    i = jnp.int32(0x5F3759DF) - lax.shift_right_arithmetic(i, 1)
    x = lax.bitcast_convert_type(i, jnp.float32)
    for _ in range(3):
        x = x * (1.5 - 0.5 * y * x * x)
    return x


def _gmf_body(job_hbm, geek_hbm, jemb_hbm, gemb_hbm, w_hbm, out_hbm,
              idx_j, idx_g, rows_j, rows_g, w_v, out_v, sem):
    wid = lax.axis_index("s") * NC + lax.axis_index("c")
    base = wid * BPW

    pltpu.sync_copy(w_hbm, w_v)
    pltpu.sync_copy(job_hbm.at[pl.ds(base, BPW)], idx_j)
    pltpu.sync_copy(geek_hbm.at[pl.ds(base, BPW)], idx_g)

    wv = [w_v[pl.ds(c * 16, 16)] for c in range(D // 16)]
    lane = lax.iota(jnp.int32, 16)
    perms = [jnp.bitwise_xor(lane, s) for s in (8, 4, 2, 1)]

    gdn = lax.GatherDimensionNumbers(
        offset_dims=(), collapsed_slice_dims=(0,), start_index_map=(0,))

    def lanesum(v):
        for p in perms:
            v = v + lax.gather(v, p[:, None], gdn, (1,),
                               mode=lax.GatherScatterMode.PROMISE_IN_BOUNDS)
        return v

    def one_pass(p, pcarry):
        pbase = p * PB

        def fire_j(g, carry):
            row0 = g * 16
            jv = idx_j[pl.ds(pbase + row0, 16)]
            for ln in range(16):
                pltpu.make_async_copy(
                    jemb_hbm.at[jv[ln]], rows_j.at[row0 + ln], sem).start()
            return carry

        def fire_g(g, carry):
            row0 = g * 16
            gv = idx_g[pl.ds(pbase + row0, 16)]
            for ln in range(16):
                pltpu.make_async_copy(
                    gemb_hbm.at[gv[ln]], rows_g.at[row0 + ln], sem).start()
            return carry

        lax.fori_loop(0, PB // 16, fire_j, 0)
        lax.fori_loop(0, PB // 16, fire_g, 0)
        pltpu.make_async_copy(jemb_hbm.at[pl.ds(0, PB)], rows_j, sem).wait()
        pltpu.make_async_copy(gemb_hbm.at[pl.ds(0, PB)], rows_g, sem).wait()

        def block(blk, carry):
            row0 = blk * BLK
            vx = jnp.zeros((16,), jnp.float32)
            vjj = jnp.zeros((16,), jnp.float32)
            vgg = jnp.zeros((16,), jnp.float32)
            for r in range(BLK):
                row = row0 + r
                px = jnp.zeros((16,), jnp.float32)
                pjj = jnp.zeros((16,), jnp.float32)
                pgg = jnp.zeros((16,), jnp.float32)
                for c in range(D // 16):
                    jv = rows_j[row, pl.ds(c * 16, 16)]
                    gv = rows_g[row, pl.ds(c * 16, 16)]
                    px = px + jv * gv * wv[c]
                    pjj = pjj + jv * jv
                    pgg = pgg + gv * gv
                m = lane == r
                vx = jnp.where(m, lanesum(px), vx)
                vjj = jnp.where(m, lanesum(pjj), vjj)
                vgg = jnp.where(m, lanesum(pgg), vgg)
            out_v[pl.ds(pbase + row0, BLK)] = vx * _nrsqrt(vjj * vgg)
            return carry

        lax.fori_loop(0, PB // BLK, block, 0)
        return pcarry

    lax.fori_loop(0, NPASS, one_pass, 0)
    pltpu.sync_copy(out_v, out_hbm.at[pl.ds(base, BPW)])


@functools.partial(jax.jit, static_argnums=())
def _gmf(job_f, geek_f, job_emb, geek_emb, w_f):
    mesh = plsc.VectorSubcoreMesh(core_axis_name="c", subcore_axis_name="s")
    run = functools.partial(
        pl.kernel,
        out_type=jax.ShapeDtypeStruct((B,), jnp.float32),
        mesh=mesh,
        scratch_types=[
            pltpu.VMEM((BPW,), jnp.int32),
            pltpu.VMEM((BPW,), jnp.int32),
            pltpu.VMEM((PB, D), jnp.float32),
            pltpu.VMEM((PB, D), jnp.float32),
            pltpu.VMEM((D,), jnp.float32),
            pltpu.VMEM((BPW,), jnp.float32),
            pltpu.SemaphoreType.DMA,
        ],
    )(_gmf_body)
    return run(job_f, geek_f, job_emb, geek_emb, w_f)


def kernel(job, geek, job_emb, geek_emb, W):
    job_f = job.reshape(-1).astype(jnp.int32)
    geek_f = geek.reshape(-1).astype(jnp.int32)
    w_f = W.reshape(-1)
    out = _gmf(job_f, geek_f, job_emb, geek_emb, w_f)
    return out.reshape(B, 1)


# speedup vs baseline: 1.5746x; 1.0018x over previous
import functools

import jax
import jax.numpy as jnp
from jax import lax
from jax.experimental import pallas as pl
from jax.experimental.pallas import tpu as pltpu
from jax.experimental.pallas import tpu_sc as plsc

B = 16384
D = 64
NC = 2
NS = 16
NW = NC * NS
BPW = B // NW
NPASS = 2
PB = BPW // NPASS
BLK = 16


def _nrsqrt(y):
    i = lax.bitcast_convert_type(y, jnp.int32)
    i = jnp.int32(0x5F3759DF) - lax.shift_right_arithmetic(i, 1)
    x = lax.bitcast_convert_type(i, jnp.float32)
    for _ in range(3):
        x = x * (1.5 - 0.5 * y * x * x)
    return x


def _gmf_body(job_hbm, geek_hbm, jemb_hbm, gemb_hbm, w_hbm, out_hbm,
              idx_j, idx_g, rows_j, rows_g, w_v, out_v, sem):
    wid = lax.axis_index("s") * NC + lax.axis_index("c")
    base = wid * BPW

    pltpu.sync_copy(w_hbm, w_v)
    pltpu.sync_copy(job_hbm.at[pl.ds(base, BPW)], idx_j)
    pltpu.sync_copy(geek_hbm.at[pl.ds(base, BPW)], idx_g)

    wv = [w_v[pl.ds(c * 16, 16)] for c in range(D // 16)]
    lane = lax.iota(jnp.int32, 16)
    perms = [jnp.bitwise_xor(lane, s) for s in (8, 4, 2, 1)]

    gdn = lax.GatherDimensionNumbers(
        offset_dims=(), collapsed_slice_dims=(0,), start_index_map=(0,))

    def lanesum(v):
        for p in perms:
            v = v + lax.gather(v, p[:, None], gdn, (1,),
                               mode=lax.GatherScatterMode.PROMISE_IN_BOUNDS)
        return v

    def one_pass(p, pcarry):
        pbase = p * PB

        @plsc.parallel_loop(0, PB // 16)
        def fire_j(g):
            row0 = g * 16
            jv = idx_j[pl.ds(pbase + row0, 16)]
            for ln in range(16):
                pltpu.make_async_copy(
                    jemb_hbm.at[jv[ln]], rows_j.at[row0 + ln], sem).start()

        @plsc.parallel_loop(0, PB // 16)
        def fire_g(g):
            row0 = g * 16
            gv = idx_g[pl.ds(pbase + row0, 16)]
            for ln in range(16):
                pltpu.make_async_copy(
                    gemb_hbm.at[gv[ln]], rows_g.at[row0 + ln], sem).start()
        pltpu.make_async_copy(jemb_hbm.at[pl.ds(0, PB)], rows_j, sem).wait()
        pltpu.make_async_copy(gemb_hbm.at[pl.ds(0, PB)], rows_g, sem).wait()

        def block(blk, carry):
            row0 = blk * BLK
            vx = jnp.zeros((16,), jnp.float32)
            vjj = jnp.zeros((16,), jnp.float32)
            vgg = jnp.zeros((16,), jnp.float32)
            for r in range(BLK):
                row = row0 + r
                px = jnp.zeros((16,), jnp.float32)
                pjj = jnp.zeros((16,), jnp.float32)
                pgg = jnp.zeros((16,), jnp.float32)
                for c in range(D // 16):
                    jv = rows_j[row, pl.ds(c * 16, 16)]
                    gv = rows_g[row, pl.ds(c * 16, 16)]
                    px = px + jv * gv * wv[c]
                    pjj = pjj + jv * jv
                    pgg = pgg + gv * gv
                m = lane == r
                vx = jnp.where(m, lanesum(px), vx)
                vjj = jnp.where(m, lanesum(pjj), vjj)
                vgg = jnp.where(m, lanesum(pgg), vgg)
            out_v[pl.ds(pbase + row0, BLK)] = vx * _nrsqrt(vjj * vgg)
            return carry

        lax.fori_loop(0, PB // BLK, block, 0)
        return pcarry

    lax.fori_loop(0, NPASS, one_pass, 0)
    pltpu.sync_copy(out_v, out_hbm.at[pl.ds(base, BPW)])


@functools.partial(jax.jit, static_argnums=())
def _gmf(job_f, geek_f, job_emb, geek_emb, w_f):
    mesh = plsc.VectorSubcoreMesh(core_axis_name="c", subcore_axis_name="s")
    run = functools.partial(
        pl.kernel,
        out_type=jax.ShapeDtypeStruct((B,), jnp.float32),
        mesh=mesh,
        scratch_types=[
            pltpu.VMEM((BPW,), jnp.int32),
            pltpu.VMEM((BPW,), jnp.int32),
            pltpu.VMEM((PB, D), jnp.float32),
            pltpu.VMEM((PB, D), jnp.float32),
            pltpu.VMEM((D,), jnp.float32),
            pltpu.VMEM((BPW,), jnp.float32),
            pltpu.SemaphoreType.DMA,
        ],
    )(_gmf_body)
    return run(job_f, geek_f, job_emb, geek_emb, w_f)


def kernel(job, geek, job_emb, geek_emb, W):
    job_f = job.reshape(-1).astype(jnp.int32)
    geek_f = geek.reshape(-1).astype(jnp.int32)
    w_f = W.reshape(-1)
    out = _gmf(job_f, geek_f, job_emb, geek_emb, w_f)
    return out.reshape(B, 1)
